# Initial kernel scaffold; baseline (speedup 1.0000x reference)
import jax
import jax.numpy as jnp
from jax.experimental import pallas as pl


def kernel(images, features, img_metas, conv_w, conv_b, cls_w, cls_b, bbox_w, bbox_b):
    raise NotImplementedError("write your pallas kernel here")


# speedup vs baseline: 5.8700x; 5.8700x over previous
import jax
import jax.numpy as jnp
import numpy as np
from jax.experimental import pallas as pl
from jax.experimental.pallas import tpu as pltpu

N = 2
C_IN = 256
C_MID = 256
H = 50
W = 84
STRIDE = 16
SCALES = (32.0, 64.0, 128.0, 256.0, 512.0)
RATIOS = (0.5, 1.0, 2.0)
A = len(SCALES) * len(RATIOS)
IMG_W = 1344
IMG_H = 800
PRE_NMS = 6000
POST_NMS = 1000
NMS_THRESH = 0.7
BBOX_XFORM_CLIP = float(np.log(1000.0 / 16.0))

NUM = A * H * W
LANES = 128
ROWS = 496
PADN = ROWS * LANES
OUT_ROWS = 1024

_NEG = -1e10


def _anchor_planes():
    scales = jnp.asarray(SCALES, jnp.float32)
    ratios = jnp.asarray(RATIOS, jnp.float32)
    h_ratios = jnp.sqrt(ratios)
    w_ratios = 1.0 / h_ratios
    ws = (w_ratios[:, None] * scales[None, :]).reshape(-1)
    hs = (h_ratios[:, None] * scales[None, :]).reshape(-1)
    base = jnp.stack([-ws, -hs, ws, hs], axis=1) / 2.0
    sx = jnp.arange(W, dtype=jnp.float32) * STRIDE
    sy = jnp.arange(H, dtype=jnp.float32) * STRIDE
    yy, xx = jnp.meshgrid(sy, sx, indexing="ij")
    shifts = jnp.stack([xx.reshape(-1), yy.reshape(-1), xx.reshape(-1), yy.reshape(-1)], axis=1)
    anchors = (shifts[:, None, :] + base[None, :, :]).reshape(-1, 4)
    wa = anchors[:, 2] - anchors[:, 0]
    ha = anchors[:, 3] - anchors[:, 1]
    cxa = anchors[:, 0] + 0.5 * wa
    cya = anchors[:, 1] + 0.5 * ha
    out = []
    for v in (wa, ha, cxa, cya):
        out.append(jnp.pad(v, (0, PADN - NUM)).reshape(ROWS, LANES))
    return out


def _nms_kernel(keys_ref, dx_ref, dy_ref, dw_ref, dh_ref,
                wa_ref, ha_ref, cxa_ref, cya_ref,
                out_ref,
                sw_ref, x1_ref, y1_ref, x2_ref, y2_ref, a2_ref):
    lg = keys_ref[0, :, :]
    idx = jax.lax.broadcasted_iota(jnp.int32, (ROWS, LANES), 0) * LANES + \
        jax.lax.broadcasted_iota(jnp.int32, (ROWS, LANES), 1)

    u = jax.lax.bitcast_convert_type(lg, jnp.int32)
    key = jnp.where(u >= 0, u, jnp.int32(-2147483648) - u)

    def tbody(_, lohi):
        lo, hi = lohi
        mid = (lo >> 1) + (hi >> 1) + (lo & hi & 1)
        cnt = jnp.sum((key >= mid).astype(jnp.int32))
        big = cnt >= PRE_NMS
        return (jnp.where(big, mid, lo), jnp.where(big, hi, mid))

    lo, _ = jax.lax.fori_loop(
        0, 32, tbody, (jnp.int32(-2147483648), jnp.int32(2147483647)))
    tie = key == lo
    need = PRE_NMS - jnp.sum((key > lo).astype(jnp.int32))

    def ibody(_, lohi):
        ilo, ihi = lohi
        mid = (ilo + ihi) // 2
        cnt = jnp.sum((tie & (idx < mid)).astype(jnp.int32))
        small = cnt < need
        return (jnp.where(small, mid, ilo), jnp.where(small, ihi, mid))

    _, istar = jax.lax.fori_loop(0, 17, ibody, (jnp.int32(0), jnp.int32(PADN)))
    sel = (key > lo) | (tie & (idx < istar))

    wa = wa_ref[...]
    ha = ha_ref[...]
    dw = jnp.minimum(dw_ref[0, :, :], BBOX_XFORM_CLIP)
    dh = jnp.minimum(dh_ref[0, :, :], BBOX_XFORM_CLIP)
    pcx = dx_ref[0, :, :] * wa + cxa_ref[...]
    pcy = dy_ref[0, :, :] * ha + cya_ref[...]
    pw = jnp.exp(dw) * wa
    ph = jnp.exp(dh) * ha
    x1 = jnp.clip(pcx - 0.5 * pw, 0.0, float(IMG_W))
    y1 = jnp.clip(pcy - 0.5 * ph, 0.0, float(IMG_H))
    x2 = jnp.clip(pcx + 0.5 * pw, 0.0, float(IMG_W))
    y2 = jnp.clip(pcy + 0.5 * ph, 0.0, float(IMG_H))
    keep = ((x2 - x1) >= 0.0) & ((y2 - y1) >= 0.0)

    sw_ref[...] = jnp.where(sel & keep, lg, _NEG)
    x1_ref[...] = x1
    y1_ref[...] = y1
    x2_ref[...] = x2
    y2_ref[...] = y2
    a2_ref[...] = (x2 - x1) * (y2 - y1)

    lane = jax.lax.broadcasted_iota(jnp.int32, (1, LANES), 1)

    def body(i, carry):
        sw = sw_ref[...]
        m = jnp.max(sw)
        valid = m > -1e9
        istar = jnp.min(jnp.where(sw == m, idx, jnp.int32(PADN)))
        oh = idx == istar
        ohf = oh.astype(jnp.float32)
        bx1 = jnp.sum(x1_ref[...] * ohf)
        by1 = jnp.sum(y1_ref[...] * ohf)
        bx2 = jnp.sum(x2_ref[...] * ohf)
        by2 = jnp.sum(y2_ref[...] * ohf)
        ba = (bx2 - bx1) * (by2 - by1)
        iw = jnp.maximum(jnp.minimum(bx2, x2_ref[...]) - jnp.maximum(bx1, x1_ref[...]), 0.0)
        ih = jnp.maximum(jnp.minimum(by2, y2_ref[...]) - jnp.maximum(by1, y1_ref[...]), 0.0)
        inter = iw * ih
        iou = inter / (ba + a2_ref[...] - inter + 1e-9)
        sup = (iou > NMS_THRESH) | oh
        sw_ref[...] = jnp.where(sup & valid, _NEG, sw)
        vf = jnp.where(valid, 1.0, 0.0)
        row = jnp.where(lane == 0, jnp.where(valid, bx1, 0.0),
              jnp.where(lane == 1, jnp.where(valid, by1, 0.0),
              jnp.where(lane == 2, jnp.where(valid, bx2, 0.0),
              jnp.where(lane == 3, jnp.where(valid, by2, 0.0),
              jnp.where(lane == 4, vf, 0.0)))))
        out_ref[0, pl.ds(i, 1), :] = row
        return carry

    jax.lax.fori_loop(0, POST_NMS, body, 0)


def _run_nms(logits_flat, breg_flat):
    padk = jnp.pad(logits_flat, ((0, 0), (0, PADN - NUM)),
                   constant_values=_NEG).reshape(N, ROWS, LANES)
    regs = []
    for j in range(4):
        regs.append(jnp.pad(breg_flat[:, :, j], ((0, 0), (0, PADN - NUM))
                            ).reshape(N, ROWS, LANES))
    wa, ha, cxa, cya = _anchor_planes()

    img_spec = pl.BlockSpec((1, ROWS, LANES), lambda i: (i, 0, 0))
    cst_spec = pl.BlockSpec((ROWS, LANES), lambda i: (0, 0))
    out = pl.pallas_call(
        _nms_kernel,
        grid=(N,),
        in_specs=[img_spec] * 5 + [cst_spec] * 4,
        out_specs=pl.BlockSpec((1, OUT_ROWS, LANES), lambda i: (i, 0, 0)),
        out_shape=jax.ShapeDtypeStruct((N, OUT_ROWS, LANES), jnp.float32),
        scratch_shapes=[pltpu.VMEM((ROWS, LANES), jnp.float32)] * 6,
        compiler_params=pltpu.CompilerParams(
            dimension_semantics=("arbitrary",)),
    )(padk, *regs, wa, ha, cxa, cya)

    boxes = out[:, :POST_NMS, :4]
    valid = out[:, :POST_NMS, 4] > 0.5
    return boxes, valid


def _conv2d(x, w, b):
    y = jax.lax.conv_general_dilated(
        x, w, (1, 1), "SAME", dimension_numbers=("NCHW", "OIHW", "NCHW"))
    return y + b[None, :, None, None]


def kernel(images, features, img_metas, conv_w, conv_b, cls_w, cls_b, bbox_w, bbox_b):
    t = jax.nn.relu(_conv2d(features, conv_w, conv_b))
    logits = _conv2d(t, cls_w, cls_b)
    bbox_reg = _conv2d(t, bbox_w, bbox_b)

    logits_flat = logits.transpose(0, 2, 3, 1).reshape(N, NUM)
    breg_flat = bbox_reg.transpose(0, 2, 3, 1).reshape(N, NUM, 4)

    scores_flat = jax.nn.sigmoid(logits_flat)
    boxes, valid = _run_nms(scores_flat, breg_flat)
    return boxes, valid, logits
